# Initial kernel scaffold; baseline (speedup 1.0000x reference)
import functools

import jax
import jax.numpy as jnp
from jax import lax
from jax.experimental import pallas as pl
from jax.experimental.pallas import tpu as pltpu
from jax.experimental.pallas import tpu_sc as plsc

D_ = 2048
B_ = 4 * 8192
NW_ = 32
BPW_ = B_ // NW_
CH_ = 32
NCHUNK_ = BPW_ // CH_


def _make_gather():
    mesh = plsc.VectorSubcoreMesh(core_axis_name="c", subcore_axis_name="s")

    @functools.partial(
        pl.kernel,
        mesh=mesh,
        out_type=jax.ShapeDtypeStruct((B_, D_), jnp.float32),
        scratch_types=[
            pltpu.VMEM((BPW_,), jnp.int32),
            pltpu.VMEM((CH_, D_), jnp.float32),
            pltpu.SemaphoreType.DMA,
        ],
    )
    def gather_kernel(idx_hbm, table_hbm, out_hbm, idx_v, rows_v, sem):
        wid = lax.axis_index("s") * 2 + lax.axis_index("c")
        base = wid * BPW_
        pltpu.sync_copy(idx_hbm.at[pl.ds(base, BPW_)], idx_v)

        @functools.partial(pl.loop, 0, NCHUNK_)
        def _chunk(c):
            row0 = base + c * CH_
            pltpu.async_copy(
                table_hbm.at[idx_v.at[pl.ds(c * CH_, CH_)]], rows_v, sem
            ).wait()
            pltpu.sync_copy(rows_v, out_hbm.at[pl.ds(row0, CH_)])

    return gather_kernel


_gather = _make_gather()


@jax.jit
def kernel(indices, table):
    flat_idx = jnp.reshape(indices, (B_,)).astype(jnp.int32)
    out = _gather(flat_idx, table)
    return jnp.reshape(out, (indices.shape[0], indices.shape[1], D_))


# speedup vs baseline: 1.4872x; 1.4872x over previous
import functools

import jax
import jax.numpy as jnp
from jax import lax
from jax.experimental import pallas as pl
from jax.experimental.pallas import tpu as pltpu
from jax.experimental.pallas import tpu_sc as plsc

D_ = 2048
B_ = 4 * 8192
NW_ = 32
BPW_ = B_ // NW_
CH_ = 32
NCHUNK_ = BPW_ // CH_


def _make_gather():
    mesh = plsc.VectorSubcoreMesh(core_axis_name="c", subcore_axis_name="s")

    @functools.partial(
        pl.kernel,
        mesh=mesh,
        out_type=jax.ShapeDtypeStruct((B_, D_), jnp.float32),
        scratch_types=[
            pltpu.VMEM((BPW_,), jnp.int32),
            pltpu.VMEM((CH_, D_), jnp.float32),
            pltpu.SemaphoreType.DMA,
        ],
    )
    def gather_kernel(idx_hbm, table_hbm, out_hbm, idx_v, rows_v, sem):
        wid = lax.axis_index("s") * 2 + lax.axis_index("c")
        base = wid * BPW_
        pltpu.sync_copy(idx_hbm.at[pl.ds(base, BPW_)], idx_v)

        @pl.loop(0, NCHUNK_)
        def _chunk(c):
            row0 = base + c * CH_
            pltpu.async_copy(
                table_hbm.at[idx_v.at[pl.ds(c * CH_, CH_)]], rows_v, sem
            ).wait()
            pltpu.sync_copy(rows_v, out_hbm.at[pl.ds(row0, CH_)])

    return gather_kernel


_gather = _make_gather()


@jax.jit
def kernel(indices, table):
    flat_idx = jnp.reshape(indices, (B_,)).astype(jnp.int32)
    out = _gather(flat_idx, table)
    return jnp.reshape(out, (indices.shape[0], indices.shape[1], D_))


# speedup vs baseline: 1.6110x; 1.0832x over previous
import functools

import jax
import jax.numpy as jnp
from jax import lax
from jax.experimental import pallas as pl
from jax.experimental.pallas import tpu as pltpu
from jax.experimental.pallas import tpu_sc as plsc

D_ = 2048
B_ = 4 * 8192
NW_ = 32
BPW_ = B_ // NW_
CH_ = 8
NCHUNK_ = BPW_ // CH_
NBUF_ = 4


def _make_gather():
    mesh = plsc.VectorSubcoreMesh(core_axis_name="c", subcore_axis_name="s")

    @functools.partial(
        pl.kernel,
        mesh=mesh,
        out_type=jax.ShapeDtypeStruct((B_, D_), jnp.float32),
        scratch_types=[
            pltpu.VMEM((BPW_,), jnp.int32),
            pltpu.VMEM((NBUF_, CH_, D_), jnp.float32),
            pltpu.SemaphoreType.DMA((NBUF_,)),
            pltpu.SemaphoreType.DMA((NBUF_,)),
        ],
    )
    def gather_kernel(idx_hbm, table_hbm, out_hbm, idx_v, rows_v, gsem, ssem):
        wid = lax.axis_index("s") * 2 + lax.axis_index("c")
        base = wid * BPW_
        pltpu.sync_copy(idx_hbm.at[pl.ds(base, BPW_)], idx_v)

        def gather_src(c):
            return table_hbm.at[idx_v.at[pl.ds(c * CH_, CH_)]]

        def out_dst(c):
            return out_hbm.at[pl.ds(base + c * CH_, CH_)]

        for b in range(NBUF_):
            pltpu.async_copy(gather_src(b), rows_v.at[b], gsem.at[b])

        def drain_and_store(c, b):
            pltpu.make_async_copy(gather_src(c), rows_v.at[b],
                                  gsem.at[b]).wait()
            pltpu.async_copy(rows_v.at[b], out_dst(c), ssem.at[b])
            pltpu.make_async_copy(rows_v.at[b], out_dst(c),
                                  ssem.at[b]).wait()

        @pl.loop(0, NCHUNK_ - NBUF_, step=NBUF_)
        def _grp(c0):
            for b in range(NBUF_):
                c = c0 + b
                drain_and_store(c, b)
                pltpu.async_copy(gather_src(c + NBUF_), rows_v.at[b],
                                 gsem.at[b])

        for b in range(NBUF_):
            drain_and_store(NCHUNK_ - NBUF_ + b, b)

    return gather_kernel


_gather = _make_gather()


@jax.jit
def kernel(indices, table):
    flat_idx = jnp.reshape(indices, (B_,)).astype(jnp.int32)
    out = _gather(flat_idx, table)
    return jnp.reshape(out, (indices.shape[0], indices.shape[1], D_))
